# Initial kernel scaffold; baseline (speedup 1.0000x reference)
import jax
import jax.numpy as jnp
from jax.experimental import pallas as pl


def kernel(dec_output, scores, gen_seq, step):
    raise NotImplementedError("write your pallas kernel here")


# speedup vs baseline: 1.6104x; 1.6104x over previous
import jax
import jax.numpy as jnp
from jax import lax
from jax.experimental import pallas as pl
from jax.experimental.pallas import tpu as pltpu

BEAM = 16
VOCAB = 1_000_000
LANES = 128
BLK_ROWS = 64
NBLK = 123
ROWS = NBLK * BLK_ROWS
PADV = ROWS * LANES - VOCAB
SEQ = 2048
EOS = 2
IBIG = 0x7FFFFFFF


def _stage1_body(d_ref, vals_ref, idx_ref, m_scr, i_scr):
    r_iota = lax.broadcasted_iota(jnp.int32, (BLK_ROWS, LANES), 0)
    l_iota = lax.broadcasted_iota(jnp.int32, (BLK_ROWS, LANES), 1)

    def s1(g, carry):
        blk = d_ref[0, pl.ds(g * BLK_ROWS, BLK_ROWS), :]
        vocab = (g * BLK_ROWS + r_iota) * LANES + l_iota
        bmax = jnp.max(blk, axis=0)
        bidx = jnp.min(jnp.where(blk == bmax[None, :], vocab, IBIG), axis=0)
        m_scr[pl.ds(g, 1), :] = bmax[None, :]
        i_scr[pl.ds(g, 1), :] = bidx[None, :]
        return carry

    lax.fori_loop(0, NBLK, s1, 0)

    M = m_scr[:, :]
    MI = i_scr[:, :]
    g_io = lax.broadcasted_iota(jnp.int32, (NBLK, LANES), 0)
    l_io = lax.broadcasted_iota(jnp.int32, (NBLK, LANES), 1)
    k_io = lax.broadcasted_iota(jnp.int32, (1, BEAM), 1)
    vals_acc = jnp.zeros((1, BEAM), jnp.float32)
    idx_acc = jnp.zeros((1, BEAM), jnp.int32)

    for k in range(BEAM):
        m = jnp.max(M)
        v_idx = jnp.min(jnp.where(M == m, MI, IBIG))
        vals_acc = jnp.where(k_io == k, m, vals_acc)
        idx_acc = jnp.where(k_io == k, v_idx, idx_acc)
        if k == BEAM - 1:
            break
        g_star = v_idx // (BLK_ROWS * LANES)
        l_star = lax.rem(v_idx, LANES)
        blk = d_ref[0, pl.ds(g_star * BLK_ROWS, BLK_ROWS), :]
        vocab = (g_star * BLK_ROWS + r_iota) * LANES + l_iota
        ok = (blk < m) | ((blk == m) & (vocab > v_idx))
        vm = jnp.where(ok, blk, -1.0)
        nmax = jnp.max(vm, axis=0)
        nidx = jnp.min(jnp.where((vm == nmax[None, :]) & ok, vocab, IBIG), axis=0)
        upd = (g_io == g_star) & (l_io == l_star)
        M = jnp.where(upd, jnp.broadcast_to(nmax[None, :], M.shape), M)
        MI = jnp.where(upd, jnp.broadcast_to(nidx[None, :], MI.shape), MI)

    vals_ref[0] = vals_acc
    idx_ref[0] = idx_acc


def _stage2_body(step_ref, vals_ref, idx_ref, scores_ref, gen_ref,
                 out_gen, out_scores, out_lens):
    s = jnp.log(vals_ref[...]) + scores_ref[...]
    f_io = (lax.broadcasted_iota(jnp.int32, (BEAM, BEAM), 0) * BEAM
            + lax.broadcasted_iota(jnp.int32, (BEAM, BEAM), 1))
    k_io = lax.broadcasted_iota(jnp.int32, (1, BEAM), 1)
    idxs = idx_ref[...]

    scores_acc = jnp.zeros((1, BEAM), jnp.float32)
    lens_acc = jnp.zeros((1, BEAM), jnp.int32)
    picks = []
    for k in range(BEAM):
        m = jnp.max(s)
        fidx = jnp.min(jnp.where(s == m, f_io, IBIG))
        bidx = jnp.min(jnp.where(f_io == fidx, idxs, IBIG))
        r = fidx // BEAM
        picks.append((r, bidx))
        scores_acc = jnp.where(k_io == k, m, scores_acc)
        s = jnp.where(f_io == fidx, -jnp.inf, s)
    out_scores[...] = scores_acc

    st = step_ref[0, 0]
    pos = lax.broadcasted_iota(jnp.int32, (1, SEQ), 1)
    for k in range(BEAM):
        r, bidx = picks[k]
        row_src = gen_ref[pl.ds(r, 1), :]
        row_orig = gen_ref[pl.ds(k, 1), :]
        merged = jnp.where(pos < st, row_src, row_orig)
        merged = jnp.where(pos == st, bidx, merged)
        out_gen[pl.ds(k, 1), :] = merged
        sl = jnp.min(jnp.where(merged == EOS, pos + 1, SEQ))
        lens_acc = jnp.where(k_io == k, sl, lens_acc)
    out_lens[...] = lens_acc


def kernel(dec_output, scores, gen_seq, step):
    flat = dec_output.reshape(BEAM, VOCAB)
    padded = jnp.concatenate(
        [flat, jnp.full((BEAM, PADV), -1.0, jnp.float32)], axis=1)
    probs = padded.reshape(BEAM, ROWS, LANES)
    vals, idxs = pl.pallas_call(
        _stage1_body,
        grid=(BEAM,),
        in_specs=[pl.BlockSpec((1, ROWS, LANES), lambda b: (b, 0, 0))],
        out_specs=[pl.BlockSpec((1, 1, BEAM), lambda b: (b, 0, 0)),
                   pl.BlockSpec((1, 1, BEAM), lambda b: (b, 0, 0))],
        out_shape=[jax.ShapeDtypeStruct((BEAM, 1, BEAM), jnp.float32),
                   jax.ShapeDtypeStruct((BEAM, 1, BEAM), jnp.int32)],
        scratch_shapes=[pltpu.VMEM((NBLK, LANES), jnp.float32),
                        pltpu.VMEM((NBLK, LANES), jnp.int32)],
    )(probs)
    vals = vals.reshape(BEAM, BEAM)
    idxs = idxs.reshape(BEAM, BEAM)

    step_arr = jnp.asarray(step, jnp.int32).reshape(1, 1)
    scores2 = scores.reshape(BEAM, 1)
    new_gen, scores_new, seq_lens = pl.pallas_call(
        _stage2_body,
        in_specs=[pl.BlockSpec(memory_space=pltpu.SMEM),
                  pl.BlockSpec((BEAM, BEAM), lambda: (0, 0)),
                  pl.BlockSpec((BEAM, BEAM), lambda: (0, 0)),
                  pl.BlockSpec((BEAM, 1), lambda: (0, 0)),
                  pl.BlockSpec((BEAM, SEQ), lambda: (0, 0))],
        out_specs=[pl.BlockSpec((BEAM, SEQ), lambda: (0, 0)),
                   pl.BlockSpec((1, BEAM), lambda: (0, 0)),
                   pl.BlockSpec((1, BEAM), lambda: (0, 0))],
        out_shape=[jax.ShapeDtypeStruct((BEAM, SEQ), jnp.int32),
                   jax.ShapeDtypeStruct((1, BEAM), jnp.float32),
                   jax.ShapeDtypeStruct((1, BEAM), jnp.int32)],
    )(step_arr, vals, idxs, scores2, gen_seq)
    return new_gen, scores_new.reshape(BEAM), seq_lens.reshape(BEAM)


# speedup vs baseline: 2.1545x; 1.3379x over previous
import jax
import jax.numpy as jnp
from jax import lax
from jax.experimental import pallas as pl
from jax.experimental.pallas import tpu as pltpu

BEAM = 16
VOCAB = 1_000_000
CHUNK = 4096
NC = (VOCAB + CHUNK - 1) // CHUNK
NCP = 256
NSEL = 16
GPC = 8
LANES = 128
POOL_ROWS = NSEL * CHUNK // LANES
BLK_ROWS = 64
NBLK = POOL_ROWS // BLK_ROWS
SEQ = 2048
EOS = 2
IBIG = 0x7FFFFFFF


def _scan_body(d_ref, ids_ref, m_scr):
    c = pl.program_id(0)
    x = d_ref[:, 0, :]
    cio = lax.broadcasted_iota(jnp.int32, (BEAM, NCP), 1)

    @pl.when(c == 0)
    def _():
        m_scr[...] = jnp.full((BEAM, NCP), -1.0, jnp.float32)

    @pl.when(c < NC - 1)
    def _():
        mx = jnp.max(x, axis=1, keepdims=True)
        m_scr[...] = jnp.where(cio == c, mx, m_scr[...])

    @pl.when(c == NC - 1)
    def _():
        lio = lax.broadcasted_iota(jnp.int32, (BEAM, CHUNK), 1)
        xm = jnp.where(c * CHUNK + lio < VOCAB, x, -1.0)
        mx = jnp.max(xm, axis=1, keepdims=True)
        M = jnp.where(cio == c, mx, m_scr[...])
        sel = jnp.zeros((BEAM, NCP), jnp.bool_)
        for _ in range(NSEL):
            row_mx = jnp.max(M, axis=1, keepdims=True)
            cid = jnp.min(jnp.where(M == row_mx, cio, IBIG), axis=1,
                          keepdims=True)
            sel = sel | (cio == cid)
            M = jnp.where(cio == cid, -2.0, M)
        kio = lax.broadcasted_iota(jnp.int32, (BEAM, NSEL), 1)
        ids_acc = jnp.zeros((BEAM, NSEL), jnp.int32)
        for k in range(NSEL):
            cid = jnp.min(jnp.where(sel, cio, IBIG), axis=1, keepdims=True)
            ids_acc = jnp.where(kio == k,
                                jnp.broadcast_to(cid, (BEAM, NSEL)), ids_acc)
            sel = sel & (cio != cid)
        ids_ref[...] = ids_acc


def _gather_body(ids_sref, *refs):
    b = pl.program_id(0)
    h = pl.program_id(1)
    out = refs[GPC]
    lio = lax.broadcasted_iota(jnp.int32, (1, CHUNK), 1)
    for i in range(GPC):
        cid = ids_sref[b, h * GPC + i]
        x = refs[i][:, 0, :]
        x = jnp.where(cid * CHUNK + lio < VOCAB, x, -1.0)
        out[0, i, :] = x[0]


def _extract_body(ids_sref, d_ref, vals_ref, idx_ref):
    b = pl.program_id(0)
    r_iota = lax.broadcasted_iota(jnp.int32, (BLK_ROWS, LANES), 0)
    l_iota = lax.broadcasted_iota(jnp.int32, (BLK_ROWS, LANES), 1)

    mparts, iparts = [], []
    for g in range(NBLK):
        blk = d_ref[0, pl.ds(g * BLK_ROWS, BLK_ROWS), :]
        local = (g * BLK_ROWS + r_iota) * LANES + l_iota
        bmax = jnp.max(blk, axis=0)
        bidx = jnp.min(jnp.where(blk == bmax[None, :], local, IBIG), axis=0)
        mparts.append(bmax[None, :])
        iparts.append(bidx[None, :])
    M = jnp.concatenate(mparts, axis=0)
    MI = jnp.concatenate(iparts, axis=0)

    g_io = lax.broadcasted_iota(jnp.int32, (NBLK, LANES), 0)
    l_io = lax.broadcasted_iota(jnp.int32, (NBLK, LANES), 1)
    k_io = lax.broadcasted_iota(jnp.int32, (1, BEAM), 1)
    vals_acc = jnp.zeros((1, BEAM), jnp.float32)
    idx_acc = jnp.zeros((1, BEAM), jnp.int32)

    for k in range(BEAM):
        m = jnp.max(M)
        p = jnp.min(jnp.where(M == m, MI, IBIG))
        j = p // CHUNK
        vocab = ids_sref[b, j] * CHUNK + lax.rem(p, CHUNK)
        vals_acc = jnp.where(k_io == k, m, vals_acc)
        idx_acc = jnp.where(k_io == k, vocab, idx_acc)
        if k == BEAM - 1:
            break
        g_star = p // (BLK_ROWS * LANES)
        l_star = lax.rem(p, LANES)
        blk = d_ref[0, pl.ds(g_star * BLK_ROWS, BLK_ROWS), :]
        local = (g_star * BLK_ROWS + r_iota) * LANES + l_iota
        ok = (blk < m) | ((blk == m) & (local > p))
        vm = jnp.where(ok, blk, -2.0)
        nmax = jnp.max(vm, axis=0)
        nidx = jnp.min(jnp.where((vm == nmax[None, :]) & ok, local, IBIG),
                       axis=0)
        upd = (g_io == g_star) & (l_io == l_star)
        M = jnp.where(upd, jnp.broadcast_to(nmax[None, :], M.shape), M)
        MI = jnp.where(upd, jnp.broadcast_to(nidx[None, :], MI.shape), MI)

    vals_ref[0] = vals_acc
    idx_ref[0] = idx_acc


def _merge_body(step_ref, vals_ref, idx_ref, scores_ref, gen_ref,
                out_gen, out_scores, out_lens):
    s = jnp.log(vals_ref[...]) + scores_ref[...]
    f_io = (lax.broadcasted_iota(jnp.int32, (BEAM, BEAM), 0) * BEAM
            + lax.broadcasted_iota(jnp.int32, (BEAM, BEAM), 1))
    k_io = lax.broadcasted_iota(jnp.int32, (1, BEAM), 1)
    idxs = idx_ref[...]

    scores_acc = jnp.zeros((1, BEAM), jnp.float32)
    lens_acc = jnp.zeros((1, BEAM), jnp.int32)
    picks = []
    for k in range(BEAM):
        m = jnp.max(s)
        fidx = jnp.min(jnp.where(s == m, f_io, IBIG))
        bidx = jnp.min(jnp.where(f_io == fidx, idxs, IBIG))
        picks.append((fidx // BEAM, bidx))
        scores_acc = jnp.where(k_io == k, m, scores_acc)
        s = jnp.where(f_io == fidx, -jnp.inf, s)
    out_scores[...] = scores_acc

    st = step_ref[0, 0]
    pos = lax.broadcasted_iota(jnp.int32, (1, SEQ), 1)
    for k in range(BEAM):
        r, bidx = picks[k]
        row_src = gen_ref[pl.ds(r, 1), :]
        row_orig = gen_ref[pl.ds(k, 1), :]
        merged = jnp.where(pos < st, row_src, row_orig)
        merged = jnp.where(pos == st, bidx, merged)
        out_gen[pl.ds(k, 1), :] = merged
        sl = jnp.min(jnp.where(merged == EOS, pos + 1, SEQ))
        lens_acc = jnp.where(k_io == k, sl, lens_acc)
    out_lens[...] = lens_acc


def kernel(dec_output, scores, gen_seq, step):
    ids = pl.pallas_call(
        _scan_body,
        grid=(NC,),
        in_specs=[pl.BlockSpec((BEAM, 1, CHUNK), lambda c: (0, 0, c))],
        out_specs=pl.BlockSpec((BEAM, NSEL), lambda c: (0, 0)),
        out_shape=jax.ShapeDtypeStruct((BEAM, NSEL), jnp.int32),
        scratch_shapes=[pltpu.VMEM((BEAM, NCP), jnp.float32)],
    )(dec_output)

    grid_spec = pltpu.PrefetchScalarGridSpec(
        num_scalar_prefetch=1,
        grid=(BEAM, NSEL // GPC),
        in_specs=[pl.BlockSpec((1, 1, CHUNK),
                               (lambda b, h, ids_m, i=i: (b, 0, ids_m[b, h * GPC + i])))
                  for i in range(GPC)],
        out_specs=pl.BlockSpec((1, GPC, CHUNK), lambda b, h, ids_m: (b, h, 0)),
    )
    pool = pl.pallas_call(
        _gather_body,
        grid_spec=grid_spec,
        out_shape=jax.ShapeDtypeStruct((BEAM, NSEL, CHUNK), jnp.float32),
    )(ids, *([dec_output] * GPC))
    pool = pool.reshape(BEAM, POOL_ROWS, LANES)

    grid_spec_d = pltpu.PrefetchScalarGridSpec(
        num_scalar_prefetch=1,
        grid=(BEAM,),
        in_specs=[pl.BlockSpec((1, POOL_ROWS, LANES), lambda b, ids_m: (b, 0, 0))],
        out_specs=[pl.BlockSpec((1, 1, BEAM), lambda b, ids_m: (b, 0, 0)),
                   pl.BlockSpec((1, 1, BEAM), lambda b, ids_m: (b, 0, 0))],
    )
    vals, idxs = pl.pallas_call(
        _extract_body,
        grid_spec=grid_spec_d,
        out_shape=[jax.ShapeDtypeStruct((BEAM, 1, BEAM), jnp.float32),
                   jax.ShapeDtypeStruct((BEAM, 1, BEAM), jnp.int32)],
    )(ids, pool)
    vals = vals.reshape(BEAM, BEAM)
    idxs = idxs.reshape(BEAM, BEAM)

    step_arr = jnp.asarray(step, jnp.int32).reshape(1, 1)
    scores2 = scores.reshape(BEAM, 1)
    new_gen, scores_new, seq_lens = pl.pallas_call(
        _merge_body,
        in_specs=[pl.BlockSpec(memory_space=pltpu.SMEM),
                  pl.BlockSpec((BEAM, BEAM), lambda: (0, 0)),
                  pl.BlockSpec((BEAM, BEAM), lambda: (0, 0)),
                  pl.BlockSpec((BEAM, 1), lambda: (0, 0)),
                  pl.BlockSpec((BEAM, SEQ), lambda: (0, 0))],
        out_specs=[pl.BlockSpec((BEAM, SEQ), lambda: (0, 0)),
                   pl.BlockSpec((1, BEAM), lambda: (0, 0)),
                   pl.BlockSpec((1, BEAM), lambda: (0, 0))],
        out_shape=[jax.ShapeDtypeStruct((BEAM, SEQ), jnp.int32),
                   jax.ShapeDtypeStruct((1, BEAM), jnp.float32),
                   jax.ShapeDtypeStruct((1, BEAM), jnp.int32)],
    )(step_arr, vals, idxs, scores2, gen_seq)
    return new_gen, scores_new.reshape(BEAM), seq_lens.reshape(BEAM)


# speedup vs baseline: 3.7195x; 1.7264x over previous
import jax
import jax.numpy as jnp
from jax import lax
from jax.experimental import pallas as pl
from jax.experimental.pallas import tpu as pltpu

BEAM = 16
VOCAB = 1_000_000
CHUNK = 4096
NC = (VOCAB + CHUNK - 1) // CHUNK
SUBC = 32
NCB = (NC + SUBC - 1) // SUBC
NCP = 256
NSEL = 16
GPC = 8
LANES = 128
POOL_ROWS = NSEL * CHUNK // LANES
BLK_ROWS = 64
NBLK = POOL_ROWS // BLK_ROWS
SEQ = 2048
EOS = 2
IBIG = 0x7FFFFFFF


def _scan_body(d_ref, ids_ref, m_scr):
    c = pl.program_id(0)
    x = d_ref[:, 0, :]
    cio = lax.broadcasted_iota(jnp.int32, (BEAM, NCP), 1)

    @pl.when(c == 0)
    def _():
        m_scr[...] = jnp.full((BEAM, NCP), -1.0, jnp.float32)

    def chunk_maxes(xv):
        out = m_scr[...]
        for i in range(SUBC):
            mx = jnp.max(xv[:, i * CHUNK:(i + 1) * CHUNK], axis=1,
                         keepdims=True)
            out = jnp.where(cio == c * SUBC + i, mx, out)
        return out

    @pl.when(c < NCB - 1)
    def _():
        m_scr[...] = chunk_maxes(x)

    @pl.when(c == NCB - 1)
    def _():
        lio = lax.broadcasted_iota(jnp.int32, (BEAM, SUBC * CHUNK), 1)
        xm = jnp.where(c * SUBC * CHUNK + lio < VOCAB, x, -1.0)
        M = chunk_maxes(xm)
        sel = jnp.zeros((BEAM, NCP), jnp.bool_)
        for _ in range(NSEL):
            row_mx = jnp.max(M, axis=1, keepdims=True)
            cid = jnp.min(jnp.where(M == row_mx, cio, IBIG), axis=1,
                          keepdims=True)
            sel = sel | (cio == cid)
            M = jnp.where(cio == cid, -2.0, M)
        kio = lax.broadcasted_iota(jnp.int32, (BEAM, NSEL), 1)
        ids_acc = jnp.zeros((BEAM, NSEL), jnp.int32)
        for k in range(NSEL):
            cid = jnp.min(jnp.where(sel, cio, IBIG), axis=1, keepdims=True)
            ids_acc = jnp.where(kio == k,
                                jnp.broadcast_to(cid, (BEAM, NSEL)), ids_acc)
            sel = sel & (cio != cid)
        ids_ref[...] = ids_acc


def _gather_body(ids_sref, *refs):
    b = pl.program_id(0)
    h = pl.program_id(1)
    out = refs[GPC]
    lio = lax.broadcasted_iota(jnp.int32, (1, CHUNK), 1)
    for i in range(GPC):
        cid = ids_sref[b, h * GPC + i]
        x = refs[i][:, 0, :]
        x = jnp.where(cid * CHUNK + lio < VOCAB, x, -1.0)
        out[0, i, :] = x[0]


def _extract_merge_body(ids_sref, d_ref, gen_ref, scores_ref, step_ref,
                        out_gen, out_scores, out_lens, vals_scr, idx_scr):
    b = pl.program_id(0)
    r_iota = lax.broadcasted_iota(jnp.int32, (BLK_ROWS, LANES), 0)
    l_iota = lax.broadcasted_iota(jnp.int32, (BLK_ROWS, LANES), 1)
    k_io = lax.broadcasted_iota(jnp.int32, (1, BEAM), 1)

    mparts, iparts = [], []
    for g in range(NBLK):
        blk = d_ref[0, pl.ds(g * BLK_ROWS, BLK_ROWS), :]
        local = (g * BLK_ROWS + r_iota) * LANES + l_iota
        bmax = jnp.max(blk, axis=0)
        bidx = jnp.min(jnp.where(blk == bmax[None, :], local, IBIG), axis=0)
        mparts.append(bmax[None, :])
        iparts.append(bidx[None, :])
    M = jnp.concatenate(mparts, axis=0)
    MI = jnp.concatenate(iparts, axis=0)

    g_io = lax.broadcasted_iota(jnp.int32, (NBLK, LANES), 0)
    l_io = lax.broadcasted_iota(jnp.int32, (NBLK, LANES), 1)
    vals_acc = jnp.zeros((1, BEAM), jnp.float32)
    idx_acc = jnp.zeros((1, BEAM), jnp.int32)

    for k in range(BEAM):
        m = jnp.max(M)
        p = jnp.min(jnp.where(M == m, MI, IBIG))
        j = p // CHUNK
        vocab = ids_sref[b, j] * CHUNK + lax.rem(p, CHUNK)
        vals_acc = jnp.where(k_io == k, m, vals_acc)
        idx_acc = jnp.where(k_io == k, vocab, idx_acc)
        if k == BEAM - 1:
            break
        g_star = p // (BLK_ROWS * LANES)
        l_star = lax.rem(p, LANES)
        blk = d_ref[0, pl.ds(g_star * BLK_ROWS, BLK_ROWS), :]
        local = (g_star * BLK_ROWS + r_iota) * LANES + l_iota
        ok = (blk < m) | ((blk == m) & (local > p))
        vm = jnp.where(ok, blk, -2.0)
        nmax = jnp.max(vm, axis=0)
        nidx = jnp.min(jnp.where((vm == nmax[None, :]) & ok, local, IBIG),
                       axis=0)
        upd = (g_io == g_star) & (l_io == l_star)
        M = jnp.where(upd, jnp.broadcast_to(nmax[None, :], M.shape), M)
        MI = jnp.where(upd, jnp.broadcast_to(nidx[None, :], MI.shape), MI)

    vals_scr[pl.ds(b, 1), :] = vals_acc
    idx_scr[pl.ds(b, 1), :] = idx_acc

    @pl.when(b == BEAM - 1)
    def _():
        s = jnp.log(vals_scr[...]) + scores_ref[...]
        f_io = (lax.broadcasted_iota(jnp.int32, (BEAM, BEAM), 0) * BEAM
                + lax.broadcasted_iota(jnp.int32, (BEAM, BEAM), 1))
        idxs = idx_scr[...]

        scores_acc = jnp.zeros((1, BEAM), jnp.float32)
        lens_acc = jnp.zeros((1, BEAM), jnp.int32)
        picks = []
        ss = s
        for k in range(BEAM):
            m = jnp.max(ss)
            fidx = jnp.min(jnp.where(ss == m, f_io, IBIG))
            bidx = jnp.min(jnp.where(f_io == fidx, idxs, IBIG))
            picks.append((fidx // BEAM, bidx))
            scores_acc = jnp.where(k_io == k, m, scores_acc)
            ss = jnp.where(f_io == fidx, -jnp.inf, ss)
        out_scores[...] = scores_acc

        st = step_ref[0, 0]
        pos = lax.broadcasted_iota(jnp.int32, (1, SEQ), 1)
        for k in range(BEAM):
            r, bidx = picks[k]
            row_src = gen_ref[pl.ds(r, 1), :]
            row_orig = gen_ref[pl.ds(k, 1), :]
            merged = jnp.where(pos < st, row_src, row_orig)
            merged = jnp.where(pos == st, bidx, merged)
            out_gen[pl.ds(k, 1), :] = merged
            sl = jnp.min(jnp.where(merged == EOS, pos + 1, SEQ))
            lens_acc = jnp.where(k_io == k, sl, lens_acc)
        out_lens[...] = lens_acc


def kernel(dec_output, scores, gen_seq, step):
    ids = pl.pallas_call(
        _scan_body,
        grid=(NCB,),
        in_specs=[pl.BlockSpec((BEAM, 1, SUBC * CHUNK), lambda c: (0, 0, c))],
        out_specs=pl.BlockSpec((BEAM, NSEL), lambda c: (0, 0)),
        out_shape=jax.ShapeDtypeStruct((BEAM, NSEL), jnp.int32),
        scratch_shapes=[pltpu.VMEM((BEAM, NCP), jnp.float32)],
    )(dec_output)

    grid_spec = pltpu.PrefetchScalarGridSpec(
        num_scalar_prefetch=1,
        grid=(BEAM, NSEL // GPC),
        in_specs=[pl.BlockSpec((1, 1, CHUNK),
                               (lambda b, h, ids_m, i=i:
                                (b, 0, ids_m[b, h * GPC + i])))
                  for i in range(GPC)],
        out_specs=pl.BlockSpec((1, GPC, CHUNK), lambda b, h, ids_m: (b, h, 0)),
    )
    pool = pl.pallas_call(
        _gather_body,
        grid_spec=grid_spec,
        out_shape=jax.ShapeDtypeStruct((BEAM, NSEL, CHUNK), jnp.float32),
    )(ids, *([dec_output] * GPC))
    pool = pool.reshape(BEAM, POOL_ROWS, LANES)

    step_arr = jnp.asarray(step, jnp.int32).reshape(1, 1)
    scores2 = scores.reshape(BEAM, 1)
    grid_spec_d = pltpu.PrefetchScalarGridSpec(
        num_scalar_prefetch=1,
        grid=(BEAM,),
        in_specs=[pl.BlockSpec((1, POOL_ROWS, LANES),
                               lambda b, ids_m: (b, 0, 0)),
                  pl.BlockSpec((BEAM, SEQ), lambda b, ids_m: (0, 0)),
                  pl.BlockSpec((BEAM, 1), lambda b, ids_m: (0, 0)),
                  pl.BlockSpec(memory_space=pltpu.SMEM)],
        out_specs=[pl.BlockSpec((BEAM, SEQ), lambda b, ids_m: (0, 0)),
                   pl.BlockSpec((1, BEAM), lambda b, ids_m: (0, 0)),
                   pl.BlockSpec((1, BEAM), lambda b, ids_m: (0, 0))],
        scratch_shapes=[pltpu.VMEM((BEAM, BEAM), jnp.float32),
                        pltpu.VMEM((BEAM, BEAM), jnp.int32)],
    )
    new_gen, scores_new, seq_lens = pl.pallas_call(
        _extract_merge_body,
        grid_spec=grid_spec_d,
        out_shape=[jax.ShapeDtypeStruct((BEAM, SEQ), jnp.int32),
                   jax.ShapeDtypeStruct((1, BEAM), jnp.float32),
                   jax.ShapeDtypeStruct((1, BEAM), jnp.int32)],
    )(ids, pool, gen_seq, scores2, step_arr)
    return new_gen, scores_new.reshape(BEAM), seq_lens.reshape(BEAM)


# speedup vs baseline: 4.8024x; 1.2911x over previous
import jax
import jax.numpy as jnp
from jax import lax
from jax.experimental import pallas as pl
from jax.experimental.pallas import tpu as pltpu

BEAM = 16
VOCAB = 1_000_000
CHUNK = 4096
NC = (VOCAB + CHUNK - 1) // CHUNK
SUBC = 32
NCB = (NC + SUBC - 1) // SUBC
NCP = 256
NSEL = 16
GPC = 8
LANES = 128
POOL_ROWS = NSEL * CHUNK // LANES
BLK_ROWS = 64
NBLK = POOL_ROWS // BLK_ROWS
SEQ = 2048
EOS = 2
IBIG = 0x7FFFFFFF


def _scan_body(d_ref, ids_ref, m_scr):
    c = pl.program_id(0)
    x = d_ref[:, 0, :]
    cio = lax.broadcasted_iota(jnp.int32, (BEAM, NCP), 1)

    @pl.when(c == 0)
    def _():
        m_scr[...] = jnp.full((BEAM, NCP), -1.0, jnp.float32)

    def chunk_maxes(xv):
        out = m_scr[...]
        for i in range(SUBC):
            mx = jnp.max(xv[:, i * CHUNK:(i + 1) * CHUNK], axis=1,
                         keepdims=True)
            out = jnp.where(cio == c * SUBC + i, mx, out)
        return out

    @pl.when(c < NCB - 1)
    def _():
        m_scr[...] = chunk_maxes(x)

    @pl.when(c == NCB - 1)
    def _():
        lio = lax.broadcasted_iota(jnp.int32, (BEAM, SUBC * CHUNK), 1)
        xm = jnp.where(c * SUBC * CHUNK + lio < VOCAB, x, -1.0)
        M = chunk_maxes(xm)
        sel = jnp.zeros((BEAM, NCP), jnp.bool_)
        for _ in range(NSEL):
            row_mx = jnp.max(M, axis=1, keepdims=True)
            cid = jnp.min(jnp.where(M == row_mx, cio, IBIG), axis=1,
                          keepdims=True)
            sel = sel | (cio == cid)
            M = jnp.where(cio == cid, -2.0, M)
        kio = lax.broadcasted_iota(jnp.int32, (BEAM, NSEL), 1)
        ids_acc = jnp.zeros((BEAM, NSEL), jnp.int32)
        for k in range(NSEL):
            cid = jnp.min(jnp.where(sel, cio, IBIG), axis=1, keepdims=True)
            ids_acc = jnp.where(kio == k,
                                jnp.broadcast_to(cid, (BEAM, NSEL)), ids_acc)
            sel = sel & (cio != cid)
        ids_ref[...] = ids_acc


def _gather_body(ids_sref, *refs):
    b = pl.program_id(0)
    h = pl.program_id(1)
    out = refs[GPC]
    lio = lax.broadcasted_iota(jnp.int32, (1, CHUNK), 1)
    for i in range(GPC):
        cid = ids_sref[b, h * GPC + i]
        x = refs[i][:, 0, :]
        x = jnp.where(cid * CHUNK + lio < VOCAB, x, -1.0)
        out[0, i, :] = x[0]


def _extract_merge_body(ids_sref, d_ref, gen_ref, scores_ref, step_ref,
                        out_gen, out_scores, out_lens):
    r_iota = lax.broadcasted_iota(jnp.int32, (BLK_ROWS, LANES), 0)
    l_iota = lax.broadcasted_iota(jnp.int32, (BLK_ROWS, LANES), 1)
    k_io = lax.broadcasted_iota(jnp.int32, (1, BEAM), 1)
    g_io = lax.broadcasted_iota(jnp.int32, (NBLK, LANES), 0)
    l_io = lax.broadcasted_iota(jnp.int32, (NBLK, LANES), 1)

    Ms, MIs = [], []
    for b in range(BEAM):
        mparts, iparts = [], []
        for g in range(NBLK):
            blk = d_ref[b, pl.ds(g * BLK_ROWS, BLK_ROWS), :]
            local = (g * BLK_ROWS + r_iota) * LANES + l_iota
            bmax = jnp.max(blk, axis=0)
            bidx = jnp.min(jnp.where(blk == bmax[None, :], local, IBIG),
                           axis=0)
            mparts.append(bmax[None, :])
            iparts.append(bidx[None, :])
        Ms.append(jnp.concatenate(mparts, axis=0))
        MIs.append(jnp.concatenate(iparts, axis=0))

    vals_accs = [jnp.zeros((1, BEAM), jnp.float32) for _ in range(BEAM)]
    idx_accs = [jnp.zeros((1, BEAM), jnp.int32) for _ in range(BEAM)]
    for k in range(BEAM):
        for b in range(BEAM):
            M, MI = Ms[b], MIs[b]
            m = jnp.max(M)
            p = jnp.min(jnp.where(M == m, MI, IBIG))
            j = p // CHUNK
            vocab = ids_sref[b, j] * CHUNK + lax.rem(p, CHUNK)
            vals_accs[b] = jnp.where(k_io == k, m, vals_accs[b])
            idx_accs[b] = jnp.where(k_io == k, vocab, idx_accs[b])
            if k == BEAM - 1:
                continue
            g_star = p // (BLK_ROWS * LANES)
            l_star = lax.rem(p, LANES)
            blk = d_ref[b, pl.ds(g_star * BLK_ROWS, BLK_ROWS), :]
            local = (g_star * BLK_ROWS + r_iota) * LANES + l_iota
            ok = (blk < m) | ((blk == m) & (local > p))
            vm = jnp.where(ok, blk, -2.0)
            nmax = jnp.max(vm, axis=0)
            nidx = jnp.min(jnp.where((vm == nmax[None, :]) & ok, local, IBIG),
                           axis=0)
            upd = (g_io == g_star) & (l_io == l_star)
            Ms[b] = jnp.where(upd, jnp.broadcast_to(nmax[None, :], M.shape), M)
            MIs[b] = jnp.where(upd, jnp.broadcast_to(nidx[None, :], MI.shape),
                               MI)

    vals = jnp.concatenate(vals_accs, axis=0)
    idxs = jnp.concatenate(idx_accs, axis=0)

    s = jnp.log(vals) + scores_ref[...]
    f_io = (lax.broadcasted_iota(jnp.int32, (BEAM, BEAM), 0) * BEAM
            + lax.broadcasted_iota(jnp.int32, (BEAM, BEAM), 1))

    scores_acc = jnp.zeros((1, BEAM), jnp.float32)
    lens_acc = jnp.zeros((1, BEAM), jnp.int32)
    picks = []
    for k in range(BEAM):
        m = jnp.max(s)
        fidx = jnp.min(jnp.where(s == m, f_io, IBIG))
        bidx = jnp.min(jnp.where(f_io == fidx, idxs, IBIG))
        picks.append((fidx // BEAM, bidx))
        scores_acc = jnp.where(k_io == k, m, scores_acc)
        s = jnp.where(f_io == fidx, -jnp.inf, s)
    out_scores[...] = scores_acc

    st = step_ref[0, 0]
    pos = lax.broadcasted_iota(jnp.int32, (1, SEQ), 1)
    for k in range(BEAM):
        r, bidx = picks[k]
        row_src = gen_ref[pl.ds(r, 1), :]
        row_orig = gen_ref[pl.ds(k, 1), :]
        merged = jnp.where(pos < st, row_src, row_orig)
        merged = jnp.where(pos == st, bidx, merged)
        out_gen[pl.ds(k, 1), :] = merged
        sl = jnp.min(jnp.where(merged == EOS, pos + 1, SEQ))
        lens_acc = jnp.where(k_io == k, sl, lens_acc)
    out_lens[...] = lens_acc


def kernel(dec_output, scores, gen_seq, step):
    ids = pl.pallas_call(
        _scan_body,
        grid=(NCB,),
        in_specs=[pl.BlockSpec((BEAM, 1, SUBC * CHUNK), lambda c: (0, 0, c))],
        out_specs=pl.BlockSpec((BEAM, NSEL), lambda c: (0, 0)),
        out_shape=jax.ShapeDtypeStruct((BEAM, NSEL), jnp.int32),
        scratch_shapes=[pltpu.VMEM((BEAM, NCP), jnp.float32)],
    )(dec_output)

    grid_spec = pltpu.PrefetchScalarGridSpec(
        num_scalar_prefetch=1,
        grid=(BEAM, NSEL // GPC),
        in_specs=[pl.BlockSpec((1, 1, CHUNK),
                               (lambda b, h, ids_m, i=i:
                                (b, 0, ids_m[b, h * GPC + i])))
                  for i in range(GPC)],
        out_specs=pl.BlockSpec((1, GPC, CHUNK), lambda b, h, ids_m: (b, h, 0)),
    )
    pool = pl.pallas_call(
        _gather_body,
        grid_spec=grid_spec,
        out_shape=jax.ShapeDtypeStruct((BEAM, NSEL, CHUNK), jnp.float32),
    )(ids, *([dec_output] * GPC))
    pool = pool.reshape(BEAM, POOL_ROWS, LANES)

    step_arr = jnp.asarray(step, jnp.int32).reshape(1, 1)
    scores2 = scores.reshape(BEAM, 1)
    grid_spec_d = pltpu.PrefetchScalarGridSpec(
        num_scalar_prefetch=1,
        grid=(1,),
        in_specs=[pl.BlockSpec((BEAM, POOL_ROWS, LANES),
                               lambda c, ids_m: (0, 0, 0)),
                  pl.BlockSpec((BEAM, SEQ), lambda c, ids_m: (0, 0)),
                  pl.BlockSpec((BEAM, 1), lambda c, ids_m: (0, 0)),
                  pl.BlockSpec(memory_space=pltpu.SMEM)],
        out_specs=[pl.BlockSpec((BEAM, SEQ), lambda c, ids_m: (0, 0)),
                   pl.BlockSpec((1, BEAM), lambda c, ids_m: (0, 0)),
                   pl.BlockSpec((1, BEAM), lambda c, ids_m: (0, 0))],
    )
    new_gen, scores_new, seq_lens = pl.pallas_call(
        _extract_merge_body,
        grid_spec=grid_spec_d,
        out_shape=[jax.ShapeDtypeStruct((BEAM, SEQ), jnp.int32),
                   jax.ShapeDtypeStruct((1, BEAM), jnp.float32),
                   jax.ShapeDtypeStruct((1, BEAM), jnp.int32)],
    )(ids, pool, gen_seq, scores2, step_arr)
    return new_gen, scores_new.reshape(BEAM), seq_lens.reshape(BEAM)


# speedup vs baseline: 6.8238x; 1.4209x over previous
import jax
import jax.numpy as jnp
from jax import lax
from jax.experimental import pallas as pl
from jax.experimental.pallas import tpu as pltpu

BEAM = 16
VOCAB = 1_000_000
CHUNK = 4096
NC = (VOCAB + CHUNK - 1) // CHUNK
SUBC = 32
NCB = (NC + SUBC - 1) // SUBC
NCP = 256
NSEL = 16
GPC = 8
LANES = 128
POOL_ROWS = NSEL * CHUNK // LANES
BLK_ROWS = 64
NBLK = POOL_ROWS // BLK_ROWS
SEQ = 2048
EOS = 2
IBIG = 0x7FFFFFFF


def _scan_body(d_ref, ids_ref, m_scr):
    c = pl.program_id(0)
    x = d_ref[:, 0, :]
    cio = lax.broadcasted_iota(jnp.int32, (BEAM, NCP), 1)

    @pl.when(c == 0)
    def _():
        m_scr[...] = jnp.full((BEAM, NCP), -1.0, jnp.float32)

    def chunk_maxes(xv):
        out = m_scr[...]
        for i in range(SUBC):
            mx = jnp.max(xv[:, i * CHUNK:(i + 1) * CHUNK], axis=1,
                         keepdims=True)
            out = jnp.where(cio == c * SUBC + i, mx, out)
        return out

    @pl.when(c < NCB - 1)
    def _():
        m_scr[...] = chunk_maxes(x)

    @pl.when(c == NCB - 1)
    def _():
        lio = lax.broadcasted_iota(jnp.int32, (BEAM, SUBC * CHUNK), 1)
        xm = jnp.where(c * SUBC * CHUNK + lio < VOCAB, x, -1.0)
        M = chunk_maxes(xm)
        sel = jnp.zeros((BEAM, NCP), jnp.bool_)
        for _ in range(NSEL):
            row_mx = jnp.max(M, axis=1, keepdims=True)
            cid = jnp.min(jnp.where(M == row_mx, cio, IBIG), axis=1,
                          keepdims=True)
            sel = sel | (cio == cid)
            M = jnp.where(cio == cid, -2.0, M)
        kio = lax.broadcasted_iota(jnp.int32, (BEAM, NSEL), 1)
        ids_acc = jnp.zeros((BEAM, NSEL), jnp.int32)
        for k in range(NSEL):
            cid = jnp.min(jnp.where(sel, cio, IBIG), axis=1, keepdims=True)
            ids_acc = jnp.where(kio == k,
                                jnp.broadcast_to(cid, (BEAM, NSEL)), ids_acc)
            sel = sel & (cio != cid)
        ids_ref[...] = ids_acc


def _gather_body(ids_sref, *refs):
    b = pl.program_id(0)
    h = pl.program_id(1)
    out = refs[GPC]
    lio = lax.broadcasted_iota(jnp.int32, (1, CHUNK), 1)
    for i in range(GPC):
        cid = ids_sref[b, h * GPC + i]
        x = refs[i][:, 0, :]
        x = jnp.where(cid * CHUNK + lio < VOCAB, x, -1.0)
        out[0, i, :] = x[0]


def _extract_merge_body(ids_sref, d_ref, ids_ref, gen_ref, scores_ref,
                        step_ref, out_gen, out_scores, out_lens, p3_scr):
    rmax = jnp.max(d_ref[...], axis=2)
    rio = lax.broadcasted_iota(jnp.int32, (BEAM, POOL_ROWS), 1)
    k_io = lax.broadcasted_iota(jnp.int32, (1, BEAM), 1)
    col_io = lax.broadcasted_iota(jnp.int32, (BEAM, BEAM), 1)

    sel = jnp.zeros((BEAM, POOL_ROWS), jnp.bool_)
    Mr = rmax
    for _ in range(BEAM):
        m = jnp.max(Mr, axis=1, keepdims=True)
        rid = jnp.min(jnp.where(Mr == m, rio, IBIG), axis=1, keepdims=True)
        sel = sel | (rio == rid)
        Mr = jnp.where(rio == rid, -2.0, Mr)
    rid_mat = jnp.zeros((BEAM, BEAM), jnp.int32)
    for k in range(BEAM):
        rid = jnp.min(jnp.where(sel, rio, IBIG), axis=1, keepdims=True)
        rid_mat = jnp.where(col_io == k,
                            jnp.broadcast_to(rid, (BEAM, BEAM)), rid_mat)
        sel = sel & (rio != rid)

    rid_masked = [jnp.where(col_io == k, rid_mat, IBIG) for k in range(BEAM)]
    for b in range(BEAM):
        for k in range(BEAM):
            rs = jnp.min(rid_masked[k][b, :])
            p3_scr[pl.ds(b, 1), pl.ds(k, 1), :] = (
                d_ref[b, pl.ds(rs, 1), :].reshape(1, 1, LANES))

    P = p3_scr[...]
    lio3 = (lax.broadcasted_iota(jnp.int32, (BEAM, BEAM, LANES), 1) * LANES
            + lax.broadcasted_iota(jnp.int32, (BEAM, BEAM, LANES), 2))
    ids_mat = ids_ref[...]

    vals = jnp.zeros((BEAM, BEAM), jnp.float32)
    idxs = jnp.zeros((BEAM, BEAM), jnp.int32)
    for k in range(BEAM):
        m = jnp.max(P, axis=(1, 2), keepdims=True)
        p = jnp.min(jnp.where(P == m, lio3, IBIG), axis=(1, 2),
                    keepdims=True)
        p2 = p[:, :, 0]
        k3 = p2 // LANES
        l3 = lax.rem(p2, LANES)
        rowid = jnp.min(jnp.where(col_io == k3, rid_mat, IBIG), axis=1,
                        keepdims=True)
        j = rowid // (CHUNK // LANES)
        cid = jnp.min(jnp.where(col_io == j, ids_mat, IBIG), axis=1,
                      keepdims=True)
        vocab = (cid * (CHUNK // LANES)
                 + lax.rem(rowid, CHUNK // LANES)) * LANES + l3
        vals = jnp.where(col_io == k, jnp.broadcast_to(m[:, :, 0], vals.shape),
                         vals)
        idxs = jnp.where(col_io == k, jnp.broadcast_to(vocab, idxs.shape),
                         idxs)
        P = jnp.where(lio3 == p, -2.0, P)

    s = jnp.log(vals) + scores_ref[...]
    f_io = (lax.broadcasted_iota(jnp.int32, (BEAM, BEAM), 0) * BEAM
            + lax.broadcasted_iota(jnp.int32, (BEAM, BEAM), 1))

    scores_acc = jnp.zeros((1, BEAM), jnp.float32)
    lens_acc = jnp.zeros((1, BEAM), jnp.int32)
    picks = []
    for k in range(BEAM):
        m = jnp.max(s)
        fidx = jnp.min(jnp.where(s == m, f_io, IBIG))
        bidx = jnp.min(jnp.where(f_io == fidx, idxs, IBIG))
        picks.append((fidx // BEAM, bidx))
        scores_acc = jnp.where(k_io == k, m, scores_acc)
        s = jnp.where(f_io == fidx, -jnp.inf, s)
    out_scores[...] = scores_acc

    st = step_ref[0, 0]
    pos = lax.broadcasted_iota(jnp.int32, (1, SEQ), 1)
    for k in range(BEAM):
        r, bidx = picks[k]
        row_src = gen_ref[pl.ds(r, 1), :]
        row_orig = gen_ref[pl.ds(k, 1), :]
        merged = jnp.where(pos < st, row_src, row_orig)
        merged = jnp.where(pos == st, bidx, merged)
        out_gen[pl.ds(k, 1), :] = merged
        sl = jnp.min(jnp.where(merged == EOS, pos + 1, SEQ))
        lens_acc = jnp.where(k_io == k, sl, lens_acc)
    out_lens[...] = lens_acc


def kernel(dec_output, scores, gen_seq, step):
    ids = pl.pallas_call(
        _scan_body,
        grid=(NCB,),
        in_specs=[pl.BlockSpec((BEAM, 1, SUBC * CHUNK), lambda c: (0, 0, c))],
        out_specs=pl.BlockSpec((BEAM, NSEL), lambda c: (0, 0)),
        out_shape=jax.ShapeDtypeStruct((BEAM, NSEL), jnp.int32),
        scratch_shapes=[pltpu.VMEM((BEAM, NCP), jnp.float32)],
    )(dec_output)

    grid_spec = pltpu.PrefetchScalarGridSpec(
        num_scalar_prefetch=1,
        grid=(BEAM, NSEL // GPC),
        in_specs=[pl.BlockSpec((1, 1, CHUNK),
                               (lambda b, h, ids_m, i=i:
                                (b, 0, ids_m[b, h * GPC + i])))
                  for i in range(GPC)],
        out_specs=pl.BlockSpec((1, GPC, CHUNK), lambda b, h, ids_m: (b, h, 0)),
    )
    pool = pl.pallas_call(
        _gather_body,
        grid_spec=grid_spec,
        out_shape=jax.ShapeDtypeStruct((BEAM, NSEL, CHUNK), jnp.float32),
    )(ids, *([dec_output] * GPC))
    pool = pool.reshape(BEAM, POOL_ROWS, LANES)

    step_arr = jnp.asarray(step, jnp.int32).reshape(1, 1)
    scores2 = scores.reshape(BEAM, 1)
    grid_spec_d = pltpu.PrefetchScalarGridSpec(
        num_scalar_prefetch=1,
        grid=(1,),
        in_specs=[pl.BlockSpec((BEAM, POOL_ROWS, LANES),
                               lambda c, ids_m: (0, 0, 0)),
                  pl.BlockSpec((BEAM, BEAM), lambda c, ids_m: (0, 0)),
                  pl.BlockSpec((BEAM, SEQ), lambda c, ids_m: (0, 0)),
                  pl.BlockSpec((BEAM, 1), lambda c, ids_m: (0, 0)),
                  pl.BlockSpec(memory_space=pltpu.SMEM)],
        out_specs=[pl.BlockSpec((BEAM, SEQ), lambda c, ids_m: (0, 0)),
                   pl.BlockSpec((1, BEAM), lambda c, ids_m: (0, 0)),
                   pl.BlockSpec((1, BEAM), lambda c, ids_m: (0, 0))],
        scratch_shapes=[pltpu.VMEM((BEAM, BEAM, LANES), jnp.float32)],
    )
    new_gen, scores_new, seq_lens = pl.pallas_call(
        _extract_merge_body,
        grid_spec=grid_spec_d,
        out_shape=[jax.ShapeDtypeStruct((BEAM, SEQ), jnp.int32),
                   jax.ShapeDtypeStruct((1, BEAM), jnp.float32),
                   jax.ShapeDtypeStruct((1, BEAM), jnp.int32)],
    )(ids, pool, ids, gen_seq, scores2, step_arr)
    return new_gen, scores_new.reshape(BEAM), seq_lens.reshape(BEAM)


# speedup vs baseline: 7.3270x; 1.0737x over previous
import jax
import jax.numpy as jnp
from jax import lax
from jax.experimental import pallas as pl
from jax.experimental.pallas import tpu as pltpu

BEAM = 16
VOCAB = 1_000_000
CHUNK = 4096
NC = (VOCAB + CHUNK - 1) // CHUNK
SUBC = 32
NCB = (NC + SUBC - 1) // SUBC
NCP = 256
NSEL = 16
GPC = 8
LANES = 128
POOL_ROWS = NSEL * CHUNK // LANES
BLK_ROWS = 64
NBLK = POOL_ROWS // BLK_ROWS
SEQ = 2048
EOS = 2
IBIG = 0x7FFFFFFF


def _scan_body(d_ref, ids_ref, m_scr):
    c = pl.program_id(0)
    x = d_ref[:, 0, :]
    cio = lax.broadcasted_iota(jnp.int32, (BEAM, NCP), 1)

    @pl.when(c == 0)
    def _():
        m_scr[...] = jnp.full((BEAM, NCP), -1.0, jnp.float32)

    def chunk_maxes(xv):
        out = m_scr[...]
        for i in range(SUBC):
            mx = jnp.max(xv[:, i * CHUNK:(i + 1) * CHUNK], axis=1,
                         keepdims=True)
            out = jnp.where(cio == c * SUBC + i, mx, out)
        return out

    @pl.when(c < NCB - 1)
    def _():
        m_scr[...] = chunk_maxes(x)

    @pl.when(c == NCB - 1)
    def _():
        lio = lax.broadcasted_iota(jnp.int32, (BEAM, SUBC * CHUNK), 1)
        xm = jnp.where(c * SUBC * CHUNK + lio < VOCAB, x, -1.0)
        M = chunk_maxes(xm)
        sel = jnp.zeros((BEAM, NCP), jnp.bool_)
        for _ in range(NSEL):
            row_mx = jnp.max(M, axis=1, keepdims=True)
            cid = jnp.min(jnp.where(M == row_mx, cio, IBIG), axis=1,
                          keepdims=True)
            sel = sel | (cio == cid)
            M = jnp.where(cio == cid, -2.0, M)
        kio = lax.broadcasted_iota(jnp.int32, (BEAM, NSEL), 1)
        ids_acc = jnp.zeros((BEAM, NSEL), jnp.int32)
        for k in range(NSEL):
            cid = jnp.min(jnp.where(sel, cio, IBIG), axis=1, keepdims=True)
            ids_acc = jnp.where(kio == k,
                                jnp.broadcast_to(cid, (BEAM, NSEL)), ids_acc)
            sel = sel & (cio != cid)
        ids_ref[...] = ids_acc


def _gather_body(ids_sref, *refs):
    b = pl.program_id(0)
    h = pl.program_id(1)
    out = refs[GPC]
    lio = lax.broadcasted_iota(jnp.int32, (1, CHUNK), 1)
    for i in range(GPC):
        cid = ids_sref[b, h * GPC + i]
        x = refs[i][:, 0, :]
        x = jnp.where(cid * CHUNK + lio < VOCAB, x, -1.0)
        out[0, pl.ds(i * (CHUNK // LANES), CHUNK // LANES), :] = (
            x.reshape(CHUNK // LANES, LANES))


def _extract_merge_body(ids_sref, d_ref, ids_ref, gen_ref, scores_ref,
                        step_ref, out_gen, out_scores, out_lens, p3_scr):
    rmax = jnp.max(d_ref[...], axis=2)
    rio = lax.broadcasted_iota(jnp.int32, (BEAM, POOL_ROWS), 1)
    k_io = lax.broadcasted_iota(jnp.int32, (1, BEAM), 1)
    col_io = lax.broadcasted_iota(jnp.int32, (BEAM, BEAM), 1)

    sel = jnp.zeros((BEAM, POOL_ROWS), jnp.bool_)
    Mr = rmax
    for _ in range(BEAM):
        m = jnp.max(Mr, axis=1, keepdims=True)
        rid = jnp.min(jnp.where(Mr == m, rio, IBIG), axis=1, keepdims=True)
        sel = sel | (rio == rid)
        Mr = jnp.where(rio == rid, -2.0, Mr)
    rid_mat = jnp.zeros((BEAM, BEAM), jnp.int32)
    for k in range(BEAM):
        rid = jnp.min(jnp.where(sel, rio, IBIG), axis=1, keepdims=True)
        rid_mat = jnp.where(col_io == k,
                            jnp.broadcast_to(rid, (BEAM, BEAM)), rid_mat)
        sel = sel & (rio != rid)

    rid_masked = [jnp.where(col_io == k, rid_mat, IBIG) for k in range(BEAM)]
    for b in range(BEAM):
        for k in range(BEAM):
            rs = jnp.min(rid_masked[k][b, :])
            p3_scr[pl.ds(b, 1), pl.ds(k, 1), :] = (
                d_ref[b, pl.ds(rs, 1), :].reshape(1, 1, LANES))

    P = p3_scr[...]
    lio3 = (lax.broadcasted_iota(jnp.int32, (BEAM, BEAM, LANES), 1) * LANES
            + lax.broadcasted_iota(jnp.int32, (BEAM, BEAM, LANES), 2))
    ids_mat = ids_ref[...]

    vals = jnp.zeros((BEAM, BEAM), jnp.float32)
    idxs = jnp.zeros((BEAM, BEAM), jnp.int32)
    for k in range(BEAM):
        m = jnp.max(P, axis=(1, 2), keepdims=True)
        p = jnp.min(jnp.where(P == m, lio3, IBIG), axis=(1, 2),
                    keepdims=True)
        p2 = p[:, :, 0]
        k3 = p2 // LANES
        l3 = lax.rem(p2, LANES)
        rowid = jnp.min(jnp.where(col_io == k3, rid_mat, IBIG), axis=1,
                        keepdims=True)
        j = rowid // (CHUNK // LANES)
        cid = jnp.min(jnp.where(col_io == j, ids_mat, IBIG), axis=1,
                      keepdims=True)
        vocab = (cid * (CHUNK // LANES)
                 + lax.rem(rowid, CHUNK // LANES)) * LANES + l3
        vals = jnp.where(col_io == k, jnp.broadcast_to(m[:, :, 0], vals.shape),
                         vals)
        idxs = jnp.where(col_io == k, jnp.broadcast_to(vocab, idxs.shape),
                         idxs)
        P = jnp.where(lio3 == p, -2.0, P)

    s = jnp.log(vals) + scores_ref[...]
    f_io = (lax.broadcasted_iota(jnp.int32, (BEAM, BEAM), 0) * BEAM
            + lax.broadcasted_iota(jnp.int32, (BEAM, BEAM), 1))

    scores_acc = jnp.zeros((1, BEAM), jnp.float32)
    lens_acc = jnp.zeros((1, BEAM), jnp.int32)
    picks = []
    for k in range(BEAM):
        m = jnp.max(s)
        fidx = jnp.min(jnp.where(s == m, f_io, IBIG))
        bidx = jnp.min(jnp.where(f_io == fidx, idxs, IBIG))
        picks.append((fidx // BEAM, bidx))
        scores_acc = jnp.where(k_io == k, m, scores_acc)
        s = jnp.where(f_io == fidx, -jnp.inf, s)
    out_scores[...] = scores_acc

    st = step_ref[0, 0]
    pos = lax.broadcasted_iota(jnp.int32, (1, SEQ), 1)
    for k in range(BEAM):
        r, bidx = picks[k]
        row_src = gen_ref[pl.ds(r, 1), :]
        row_orig = gen_ref[pl.ds(k, 1), :]
        merged = jnp.where(pos < st, row_src, row_orig)
        merged = jnp.where(pos == st, bidx, merged)
        out_gen[pl.ds(k, 1), :] = merged
        sl = jnp.min(jnp.where(merged == EOS, pos + 1, SEQ))
        lens_acc = jnp.where(k_io == k, sl, lens_acc)
    out_lens[...] = lens_acc


def kernel(dec_output, scores, gen_seq, step):
    ids = pl.pallas_call(
        _scan_body,
        grid=(NCB,),
        in_specs=[pl.BlockSpec((BEAM, 1, SUBC * CHUNK), lambda c: (0, 0, c))],
        out_specs=pl.BlockSpec((BEAM, NSEL), lambda c: (0, 0)),
        out_shape=jax.ShapeDtypeStruct((BEAM, NSEL), jnp.int32),
        scratch_shapes=[pltpu.VMEM((BEAM, NCP), jnp.float32)],
    )(dec_output)

    grid_spec = pltpu.PrefetchScalarGridSpec(
        num_scalar_prefetch=1,
        grid=(BEAM, NSEL // GPC),
        in_specs=[pl.BlockSpec((1, 1, CHUNK),
                               (lambda b, h, ids_m, i=i:
                                (b, 0, ids_m[b, h * GPC + i])))
                  for i in range(GPC)],
        out_specs=pl.BlockSpec((1, GPC * CHUNK // LANES, LANES),
                               lambda b, h, ids_m: (b, h, 0)),
    )
    pool = pl.pallas_call(
        _gather_body,
        grid_spec=grid_spec,
        out_shape=jax.ShapeDtypeStruct((BEAM, POOL_ROWS, LANES), jnp.float32),
    )(ids, *([dec_output] * GPC))

    step_arr = jnp.asarray(step, jnp.int32).reshape(1, 1)
    scores2 = scores.reshape(BEAM, 1)
    grid_spec_d = pltpu.PrefetchScalarGridSpec(
        num_scalar_prefetch=1,
        grid=(1,),
        in_specs=[pl.BlockSpec((BEAM, POOL_ROWS, LANES),
                               lambda c, ids_m: (0, 0, 0)),
                  pl.BlockSpec((BEAM, BEAM), lambda c, ids_m: (0, 0)),
                  pl.BlockSpec((BEAM, SEQ), lambda c, ids_m: (0, 0)),
                  pl.BlockSpec((BEAM, 1), lambda c, ids_m: (0, 0)),
                  pl.BlockSpec(memory_space=pltpu.SMEM)],
        out_specs=[pl.BlockSpec((BEAM, SEQ), lambda c, ids_m: (0, 0)),
                   pl.BlockSpec((1, BEAM), lambda c, ids_m: (0, 0)),
                   pl.BlockSpec((1, BEAM), lambda c, ids_m: (0, 0))],
        scratch_shapes=[pltpu.VMEM((BEAM, BEAM, LANES), jnp.float32)],
    )
    new_gen, scores_new, seq_lens = pl.pallas_call(
        _extract_merge_body,
        grid_spec=grid_spec_d,
        out_shape=[jax.ShapeDtypeStruct((BEAM, SEQ), jnp.int32),
                   jax.ShapeDtypeStruct((1, BEAM), jnp.float32),
                   jax.ShapeDtypeStruct((1, BEAM), jnp.int32)],
    )(ids, pool, ids, gen_seq, scores2, step_arr)
    return new_gen, scores_new.reshape(BEAM), seq_lens.reshape(BEAM)
